# baseline (speedup 1.0000x reference)
import functools

import jax
import jax.numpy as jnp
from jax import lax
from jax.experimental import pallas as pl
from jax.experimental.pallas import tpu as pltpu
from jax.experimental.pallas import tpu_sc as plsc

_IM_DIMX = 1333
_IM_DIMY = 800

_N_STREAMS = 4
_CHUNK = 4

_DH = 416
_DW = 768

_NPAD = 1024


def _sc_corner_scatter():
    n_sub = 16
    slab = (_DH * _DW) // n_sub
    bufs_per_tile = (4 * _NPAD) // (128 * n_sub)
    mesh = plsc.VectorSubcoreMesh(core_axis_name="c", subcore_axis_name="s")

    @functools.partial(
        pl.kernel,
        mesh=mesh,
        out_type=jax.ShapeDtypeStruct((_DH * _DW,), jnp.int32),
        scratch_types=(
            [pltpu.VMEM((128,), jnp.int32) for _ in range(2 * bufs_per_tile)]
            + [pltpu.VMEM_SHARED((_DH * _DW,), jnp.int32)]
        ),
    )
    def sc_kernel(idx_hbm, val_hbm, zero_hbm, out_hbm, *scr):
        idxv = scr[:bufs_per_tile]
        valv = scr[bufs_per_tile:2 * bufs_per_tile]
        d_sh = scr[2 * bufs_per_tile]
        cid = lax.axis_index("c")
        sid = lax.axis_index("s")

        @pl.when(cid == 0)
        def _core0():
            pltpu.sync_copy(zero_hbm.at[pl.ds(sid * slab, slab)],
                            d_sh.at[pl.ds(sid * slab, slab)])
            plsc.subcore_barrier()
            for s in range(bufs_per_tile):
                b = sid * bufs_per_tile + s
                pltpu.sync_copy(idx_hbm.at[pl.ds(b * 128, 128)], idxv[s])
                pltpu.sync_copy(val_hbm.at[pl.ds(b * 128, 128)], valv[s])
                pltpu.sync_copy(valv[s], d_sh.at[idxv[s]], add=True)
            plsc.subcore_barrier()
            pltpu.sync_copy(d_sh.at[pl.ds(sid * slab, slab)],
                            out_hbm.at[pl.ds(sid * slab, slab)])

    return sc_kernel


def _tc_corners_body(h, w, N):
    sx = float(w) / _IM_DIMX
    sy = float(h) / _IM_DIMY

    def _body(label_ref, idx_ref, val_ref):
        lbl = label_ref[...].astype(jnp.float32)
        x1 = jnp.clip(jnp.round(lbl[:, 0] * sx), 0.0, float(w - 1))
        y1 = jnp.clip(jnp.round(lbl[:, 1] * sy), 0.0, float(h - 1))
        x2 = jnp.clip(jnp.round(lbl[:, 2] * sx), 0.0, float(w))
        y2 = jnp.clip(jnp.round(lbl[:, 3] * sy), 0.0, float(h))
        valid = jnp.logical_not(
            (x2 <= x1) | (y2 <= y1) | (x1 + x2 >= float(w)) | (y1 + y2 >= float(h))
        )
        use = valid.astype(jnp.int32)
        x1i = x1.astype(jnp.int32)
        y1i = y1.astype(jnp.int32)
        x2i = x2.astype(jnp.int32)
        y2i = y2.astype(jnp.int32)
        idx_ref[...] = jnp.zeros((4 * _NPAD,), jnp.int32)
        val_ref[...] = jnp.zeros((4 * _NPAD,), jnp.int32)
        corners = (
            (y1i * _DW + x1i, use),
            (y1i * _DW + x2i, -use),
            (y2i * _DW + x1i, -use),
            (y2i * _DW + x2i, use),
        )
        for c, (idx, val) in enumerate(corners):
            idx_ref[pl.ds(c * _NPAD, N)] = idx * use
            val_ref[pl.ds(c * _NPAD, N)] = val

    return _body


def _tc_sums_body(n_streams):
    def _body(*refs):
        x_refs = refs[:n_streams]
        out_ref = refs[n_streams]
        i = pl.program_id(0)
        j = pl.program_id(1)
        s = jnp.sum(x_refs[0][...])
        for r in x_refs[1:]:
            s = s + jnp.sum(r[...])

        @pl.when(j == 0)
        def _init():
            out_ref[i] = s

        @pl.when(j != 0)
        def _accum():
            out_ref[i] = out_ref[i] + s

    return _body


def _tc_finalize_body(b, h, w, L):
    tn = float(b * h * w)

    def _body(d_ref, sums_ref, out_ref):
        d = d_ref[...].astype(jnp.float32)
        ry = lax.broadcasted_iota(jnp.int32, (h, _DH), 0)
        cy = lax.broadcasted_iota(jnp.int32, (h, _DH), 1)
        t_mat = (cy <= ry).astype(jnp.float32)
        rx = lax.broadcasted_iota(jnp.int32, (_DW, w), 0)
        cx = lax.broadcasted_iota(jnp.int32, (_DW, w), 1)
        u_mat = (rx <= cx).astype(jnp.float32)
        p = lax.dot_general(
            t_mat, d, (((1,), (0,)), ((), ())), preferred_element_type=jnp.float32
        )
        count = lax.dot_general(
            p, u_mat, (((1,), (0,)), ((), ())), preferred_element_type=jnp.float32
        )
        covered = jnp.sum((count > 0.5).astype(jnp.float32))
        pi = covered / tn
        tot = 0.0
        for k in range(L):
            tot = tot + (sums_ref[k] / tn - pi) ** 2
        out_ref[0, 0] = tot / float(L)

    return _body


def kernel(h_masks, label):
    L, B, C, H, W = h_masks.shape
    K, Nb, _ = label.shape
    N = K * Nb
    ns = _N_STREAMS
    ck = _CHUNK
    flat = jnp.reshape(h_masks, (L, B * C, H, W))
    n_j = (B * C) // (ns * ck)

    boxes = jnp.reshape(label, (N, 4)).astype(jnp.int32)
    cidx, cval = pl.pallas_call(
        _tc_corners_body(H, W, N),
        in_specs=[pl.BlockSpec(memory_space=pltpu.VMEM)],
        out_specs=[
            pl.BlockSpec(memory_space=pltpu.VMEM),
            pl.BlockSpec(memory_space=pltpu.VMEM),
        ],
        out_shape=[
            jax.ShapeDtypeStruct((4 * _NPAD,), jnp.int32),
            jax.ShapeDtypeStruct((4 * _NPAD,), jnp.int32),
        ],
    )(boxes)
    zero_d = jnp.zeros((_DH * _DW,), jnp.int32)
    d_arr = _sc_corner_scatter()(cidx, cval, zero_d)
    d_arr = jnp.reshape(d_arr, (_DH, _DW))

    specs = []
    for s in range(ns):
        specs.append(
            pl.BlockSpec((1, ck, H, W), lambda i, j, s=s: (i, j * ns + s, 0, 0))
        )
    sums = pl.pallas_call(
        _tc_sums_body(ns),
        grid=(L, n_j),
        in_specs=specs,
        out_specs=pl.BlockSpec(memory_space=pltpu.SMEM),
        out_shape=jax.ShapeDtypeStruct((L,), jnp.float32),
    )(*([flat] * ns))

    out = pl.pallas_call(
        _tc_finalize_body(B, H, W, L),
        in_specs=[
            pl.BlockSpec(memory_space=pltpu.VMEM),
            pl.BlockSpec(memory_space=pltpu.SMEM),
        ],
        out_specs=pl.BlockSpec(memory_space=pltpu.SMEM),
        out_shape=jax.ShapeDtypeStruct((1, 1), jnp.float32),
    )(d_arr, sums)
    return out[0, 0]
